# baseline (speedup 1.0000x reference)
import functools

import jax
import jax.numpy as jnp
from jax import lax
from jax.experimental import pallas as pl
from jax.experimental.pallas import tpu as pltpu
from jax.experimental.pallas import tpu_sc as plsc

B, N, K = 1, 100000, 16
IN_CH, OUT_CH = 16, 64
LAST_CH = IN_CH + 3
WN_OUT = 16
NK = N * K
TW = 32

NW = 32
CH = 512
NCHUNK = NK // CH
CPW_LO = NCHUNK // NW
REM = NCHUNK - CPW_LO * NW
NB = 4
GMAX = (CPW_LO + 1 + NB - 1) // NB


@functools.cache
def _make_sc_gather():
    mesh = plsc.VectorSubcoreMesh(core_axis_name="c", subcore_axis_name="s")

    scratch = (
        [pltpu.VMEM((CH,), jnp.int32) for _ in range(NB)]
        + [pltpu.VMEM((CH, TW), jnp.float32) for _ in range(NB)]
        + [pltpu.SemaphoreType.DMA for _ in range(3 * NB)]
    )

    @functools.partial(
        pl.kernel,
        mesh=mesh,
        out_type=jax.ShapeDtypeStruct((NK, TW), jnp.float32),
        scratch_types=scratch,
        compiler_params=pltpu.CompilerParams(use_tc_tiling_on_sc=False),
    )
    def _sc_gather(idx_hbm, tbl_hbm, out_hbm, *scr):
        idx_v = scr[0:NB]
        bt = scr[NB:2 * NB]
        isem = scr[2 * NB:3 * NB]
        gsem = scr[3 * NB:4 * NB]
        wsem = scr[4 * NB:5 * NB]

        wid = lax.axis_index("s") * 2 + lax.axis_index("c")
        nch = CPW_LO + (wid < REM).astype(jnp.int32)

        def off(j):
            return wid * CH + j * (NW * CH)

        for b in range(NB):
            pltpu.async_copy(idx_hbm.at[pl.ds(off(b), CH)], idx_v[b], isem[b])

        def body(g, carry):
            for b in range(NB):
                j = g * NB + b

                @pl.when(j < nch)
                def _():
                    @pl.when(j >= NB)
                    def _():
                        pltpu.make_async_copy(
                            bt[b], out_hbm.at[pl.ds(off(j - NB), CH)],
                            wsem[b]).wait()

                    pltpu.make_async_copy(
                        idx_hbm.at[pl.ds(off(j), CH)], idx_v[b],
                        isem[b]).wait()
                    pltpu.async_copy(tbl_hbm.at[idx_v[b]], bt[b], gsem[b])

            for b in range(NB):
                j = g * NB + b

                @pl.when(j < nch)
                def _():
                    pltpu.make_async_copy(
                        tbl_hbm.at[idx_v[b]], bt[b], gsem[b]).wait()
                    pltpu.async_copy(
                        bt[b], out_hbm.at[pl.ds(off(j), CH)], wsem[b])

                    @pl.when(j + NB < nch)
                    def _():
                        pltpu.async_copy(
                            idx_hbm.at[pl.ds(off(j + NB), CH)], idx_v[b],
                            isem[b])

            return carry

        lax.fori_loop(0, GMAX, body, 0)

        for b in range(NB):
            pltpu.make_async_copy(
                bt[b], out_hbm.at[pl.ds(off(0), CH)], wsem[b]).wait()

    return _sc_gather


def _tc_body(g_ref, q_ref, w1_ref, b1_ref, w2_ref, b2_ref,
             w3_ref, b3_ref, wl_ref, bl_ref, of_ref, ow_ref):
    P = q_ref.shape[0]
    R = P * K
    q = q_ref[...]
    loc43 = g_ref[:, :, IN_CH:IN_CH + 4] - q[None]
    loc4 = loc43.reshape(R, 4)
    h = jnp.maximum(
        jnp.dot(loc4, w1_ref[...], preferred_element_type=jnp.float32)
        + b1_ref[...], 0.0)
    h = jnp.maximum(
        jnp.dot(h, w2_ref[...], preferred_element_type=jnp.float32)
        + b2_ref[...], 0.0)
    w = jnp.maximum(
        jnp.dot(h, w3_ref[...], preferred_element_type=jnp.float32)
        + b3_ref[...], 0.0)
    gf19 = jnp.concatenate(
        [g_ref[...].reshape(R, TW)[:, :IN_CH], loc4[:, :3]], axis=1)
    CM = LAST_CH * WN_OUT
    j_c = lax.broadcasted_iota(jnp.int32, (LAST_CH, CM), 1) // WN_OUT
    row_c = lax.broadcasted_iota(jnp.int32, (LAST_CH, CM), 0)
    expand_c = (j_c == row_c).astype(jnp.float32)
    j_m = lax.broadcasted_iota(jnp.int32, (WN_OUT, CM), 1) % WN_OUT
    row_m = lax.broadcasted_iota(jnp.int32, (WN_OUT, CM), 0)
    expand_m = (j_m == row_m).astype(jnp.float32)
    grep = jnp.dot(gf19, expand_c, preferred_element_type=jnp.float32)
    wrep = jnp.dot(w, expand_m, preferred_element_type=jnp.float32)
    z = grep * wrep
    z3 = z.reshape(K, P, CM)
    c = z3[0]
    for k in range(1, K):
        c = c + z3[k]
    of_ref[...] = jnp.maximum(
        jnp.dot(c, wl_ref[...], preferred_element_type=jnp.float32)
        + bl_ref[...], 0.0)
    ow_ref[...] = loc43[:, :, :3]


def kernel(dense_xyz, dense_feats, nei_inds, W1, b1, W2, b2, W3, b3, Wl, bl):
    tbl = jnp.concatenate(
        [dense_feats[0], dense_xyz[0],
         jnp.zeros((N, TW - IN_CH - 3), jnp.float32)], axis=1)
    idx = nei_inds[0].T.reshape(NK)
    g = _make_sc_gather()(idx, tbl)
    g3 = g.reshape(K, N, TW)

    qpad = jnp.pad(dense_xyz[0], ((0, 0), (0, 1)))
    W1p = jnp.concatenate([W1, jnp.zeros((1, 8), jnp.float32)], axis=0)

    P = 400
    grid = (N // P,)
    of, ow = pl.pallas_call(
        _tc_body,
        grid=grid,
        in_specs=[
            pl.BlockSpec((K, P, TW), lambda i: (0, i, 0)),
            pl.BlockSpec((P, 4), lambda i: (i, 0)),
            pl.BlockSpec((4, 8), lambda i: (0, 0)),
            pl.BlockSpec((1, 8), lambda i: (0, 0)),
            pl.BlockSpec((8, 8), lambda i: (0, 0)),
            pl.BlockSpec((1, 8), lambda i: (0, 0)),
            pl.BlockSpec((8, 16), lambda i: (0, 0)),
            pl.BlockSpec((1, 16), lambda i: (0, 0)),
            pl.BlockSpec((LAST_CH * WN_OUT, OUT_CH), lambda i: (0, 0)),
            pl.BlockSpec((1, OUT_CH), lambda i: (0, 0)),
        ],
        out_specs=[
            pl.BlockSpec((P, OUT_CH), lambda i: (i, 0)),
            pl.BlockSpec((K, P, 3), lambda i: (0, i, 0)),
        ],
        out_shape=[
            jax.ShapeDtypeStruct((N, OUT_CH), jnp.float32),
            jax.ShapeDtypeStruct((K, N, 3), jnp.float32),
        ],
    )(g3, qpad, W1p, b1.reshape(1, 8), W2, b2.reshape(1, 8),
      W3, b3.reshape(1, 16), Wl, bl.reshape(1, OUT_CH))
    wni = ow.transpose(1, 0, 2).reshape(B, N, K, 3)
    return (of.reshape(B, N, OUT_CH), wni)


# speedup vs baseline: 1.5061x; 1.5061x over previous
import functools

import jax
import jax.numpy as jnp
import numpy as np
from jax import lax
from jax.experimental import pallas as pl
from jax.experimental.pallas import tpu as pltpu
from jax.experimental.pallas import tpu_sc as plsc

B, N, K = 1, 100000, 16
IN_CH, OUT_CH = 16, 64
LAST_CH = IN_CH + 3
WN_OUT = 16
NK = N * K
TW = 32
CM = LAST_CH * WN_OUT

NW = 32
CH = 512
NCHUNK = NK // CH
CPW_LO = NCHUNK // NW
REM = NCHUNK - CPW_LO * NW
NB = 4
GMAX = (CPW_LO + 1 + NB - 1) // NB


@functools.cache
def _make_sc_gather():
    mesh = plsc.VectorSubcoreMesh(core_axis_name="c", subcore_axis_name="s")

    scratch = (
        [pltpu.VMEM((CH,), jnp.int32) for _ in range(NB)]
        + [pltpu.VMEM((CH, TW), jnp.float32) for _ in range(NB)]
        + [pltpu.SemaphoreType.DMA for _ in range(3 * NB)]
    )

    @functools.partial(
        pl.kernel,
        mesh=mesh,
        out_type=jax.ShapeDtypeStruct((NK, TW), jnp.float32),
        scratch_types=scratch,
        compiler_params=pltpu.CompilerParams(use_tc_tiling_on_sc=False),
    )
    def _sc_gather(idx_hbm, tbl_hbm, out_hbm, *scr):
        idx_v = scr[0:NB]
        bt = scr[NB:2 * NB]
        isem = scr[2 * NB:3 * NB]
        gsem = scr[3 * NB:4 * NB]
        wsem = scr[4 * NB:5 * NB]

        wid = lax.axis_index("s") * 2 + lax.axis_index("c")
        nch = CPW_LO + (wid < REM).astype(jnp.int32)

        def off(j):
            return wid * CH + j * (NW * CH)

        for b in range(NB):
            pltpu.async_copy(idx_hbm.at[pl.ds(off(b), CH)], idx_v[b], isem[b])

        def body(g, carry):
            for b in range(NB):
                j = g * NB + b

                @pl.when(j < nch)
                def _():
                    @pl.when(j >= NB)
                    def _():
                        pltpu.make_async_copy(
                            bt[b], out_hbm.at[pl.ds(off(j - NB), CH)],
                            wsem[b]).wait()

                    pltpu.make_async_copy(
                        idx_hbm.at[pl.ds(off(j), CH)], idx_v[b],
                        isem[b]).wait()
                    pltpu.async_copy(tbl_hbm.at[idx_v[b]], bt[b], gsem[b])

            for b in range(NB):
                j = g * NB + b

                @pl.when(j < nch)
                def _():
                    pltpu.make_async_copy(
                        tbl_hbm.at[idx_v[b]], bt[b], gsem[b]).wait()
                    pltpu.async_copy(
                        bt[b], out_hbm.at[pl.ds(off(j), CH)], wsem[b])

                    @pl.when(j + NB < nch)
                    def _():
                        pltpu.async_copy(
                            idx_hbm.at[pl.ds(off(j + NB), CH)], idx_v[b],
                            isem[b])

            return carry

        lax.fori_loop(0, GMAX, body, 0)

        for b in range(NB):
            pltpu.make_async_copy(
                bt[b], out_hbm.at[pl.ds(off(0), CH)], wsem[b]).wait()

    return _sc_gather


def _np_blockdiag(blk, n):
    return np.kron(np.eye(n, dtype=np.float32), blk).astype(np.float32)


def _build_static():
    e32 = np.zeros((TW, CM), np.float32)
    for u in range(LAST_CH):
        for m in range(WN_OUT):
            e32[u, u * WN_OUT + m] = 1.0
    eq4 = np.zeros((4, CM), np.float32)
    for d in range(3):
        for m in range(WN_OUT):
            eq4[d, (IN_CH + d) * WN_OUT + m] = 1.0
    t16 = np.zeros((WN_OUT, CM), np.float32)
    for m in range(WN_OUT):
        for c in range(LAST_CH):
            t16[m, c * WN_OUT + m] = 1.0
    s32 = np.zeros((TW, 4), np.float32)
    for i in range(4):
        s32[IN_CH + i, i] = 1.0
    return (
        jnp.asarray(_np_blockdiag(e32, 4)),
        jnp.asarray(_np_blockdiag(eq4, 4)),
        jnp.asarray(_np_blockdiag(t16, 4)),
        jnp.asarray(_np_blockdiag(s32, 4)),
    )


def _tc_body(g_ref, q_ref, wt1_ref, wq1_ref, wt2_ref, wt3_ref,
             et_ref, eq_ref, tt_ref, s_ref, wlt_ref,
             b1t_ref, b2t_ref, b3t_ref, blt_ref, of_ref, ow_ref):
    P4 = q_ref.shape[0]
    Rp = K * P4
    gp = g_ref[...]
    gp2 = gp.reshape(Rp, 128)
    q = q_ref[...]

    t1 = jnp.dot(q, wq1_ref[...], preferred_element_type=jnp.float32)
    a1 = jnp.dot(gp2, wt1_ref[...], preferred_element_type=jnp.float32)
    h = jnp.maximum(a1.reshape(K, P4, 32) - t1[None] + b1t_ref[...], 0.0)
    h = h.reshape(Rp, 32)
    h = jnp.maximum(
        jnp.dot(h, wt2_ref[...], preferred_element_type=jnp.float32)
        + b2t_ref[...], 0.0)
    wp = jnp.maximum(
        jnp.dot(h, wt3_ref[...], preferred_element_type=jnp.float32)
        + b3t_ref[...], 0.0)
    wp3 = wp.reshape(K, P4, 64)

    lp = jnp.dot(gp2, s_ref[...], preferred_element_type=jnp.float32)
    ow_ref[...] = lp.reshape(K, P4, 16) - q[None]

    qe = jnp.dot(q, eq_ref[...], preferred_element_type=jnp.float32)
    acc = None
    for kg in range(4):
        gg = gp[4 * kg:4 * kg + 4].reshape(4 * P4, 128)
        wg = wp3[4 * kg:4 * kg + 4].reshape(4 * P4, 64)
        grep = jnp.dot(gg, et_ref[...],
                       preferred_element_type=jnp.float32).reshape(
                           4, P4, 4 * CM) - qe[None]
        wrep = jnp.dot(wg, tt_ref[...],
                       preferred_element_type=jnp.float32).reshape(
                           4, P4, 4 * CM)
        z = grep * wrep
        sg = (z[0] + z[1]) + (z[2] + z[3])
        acc = sg if acc is None else acc + sg

    of_ref[...] = jnp.maximum(
        jnp.dot(acc, wlt_ref[...], preferred_element_type=jnp.float32)
        + blt_ref[...], 0.0)


def kernel(dense_xyz, dense_feats, nei_inds, W1, b1, W2, b2, W3, b3, Wl, bl):
    tbl = jnp.concatenate(
        [dense_feats[0], dense_xyz[0],
         jnp.zeros((N, TW - IN_CH - 3), jnp.float32)], axis=1)
    idx = nei_inds[0].T.reshape(NK)
    g = _make_sc_gather()(idx, tbl)
    g4 = g.reshape(K, N // 4, 128)

    qpk = jnp.pad(dense_xyz[0], ((0, 0), (0, 1))).reshape(N // 4, 16)

    bd = jax.scipy.linalg.block_diag
    W1blk = jnp.concatenate(
        [jnp.zeros((IN_CH, 8), jnp.float32), W1,
         jnp.zeros((TW - IN_CH - 3, 8), jnp.float32)], axis=0)
    W1q = jnp.concatenate([W1, jnp.zeros((1, 8), jnp.float32)], axis=0)
    Wt1 = bd(W1blk, W1blk, W1blk, W1blk)
    Wq1 = bd(W1q, W1q, W1q, W1q)
    Wt2 = bd(W2, W2, W2, W2)
    Wt3 = bd(W3, W3, W3, W3)
    Wlt = bd(Wl, Wl, Wl, Wl)
    Et, Eq, Tt, S = _build_static()
    b1t = jnp.tile(b1, 4).reshape(1, 32)
    b2t = jnp.tile(b2, 4).reshape(1, 32)
    b3t = jnp.tile(b3, 4).reshape(1, 64)
    blt = jnp.tile(bl, 4).reshape(1, 256)

    P = 800
    P4 = P // 4
    grid = (N // P,)
    full = lambda i: (0, 0)
    of4, ow = pl.pallas_call(
        _tc_body,
        grid=grid,
        in_specs=[
            pl.BlockSpec((K, P4, 128), lambda i: (0, i, 0)),
            pl.BlockSpec((P4, 16), lambda i: (i, 0)),
            pl.BlockSpec((128, 32), full),
            pl.BlockSpec((16, 32), full),
            pl.BlockSpec((32, 32), full),
            pl.BlockSpec((32, 64), full),
            pl.BlockSpec((128, 4 * CM), full),
            pl.BlockSpec((16, 4 * CM), full),
            pl.BlockSpec((64, 4 * CM), full),
            pl.BlockSpec((128, 16), full),
            pl.BlockSpec((4 * CM, 256), full),
            pl.BlockSpec((1, 32), full),
            pl.BlockSpec((1, 32), full),
            pl.BlockSpec((1, 64), full),
            pl.BlockSpec((1, 256), full),
        ],
        out_specs=[
            pl.BlockSpec((P4, 256), lambda i: (i, 0)),
            pl.BlockSpec((K, P4, 16), lambda i: (0, i, 0)),
        ],
        out_shape=[
            jax.ShapeDtypeStruct((N // 4, 256), jnp.float32),
            jax.ShapeDtypeStruct((K, N // 4, 16), jnp.float32),
        ],
    )(g4, qpk, Wt1, Wq1, Wt2, Wt3, Et, Eq, Tt, S, Wlt,
      b1t, b2t, b3t, blt)
    of = of4.reshape(B, N, OUT_CH)
    wni = ow.reshape(K, N, 4)[:, :, :3].transpose(1, 0, 2).reshape(B, N, K, 3)
    return (of, wni)


# speedup vs baseline: 1.5800x; 1.0491x over previous
import functools

import jax
import jax.numpy as jnp
import numpy as np
from jax import lax
from jax.experimental import pallas as pl
from jax.experimental.pallas import tpu as pltpu
from jax.experimental.pallas import tpu_sc as plsc

B, N, K = 1, 100000, 16
IN_CH, OUT_CH = 16, 64
LAST_CH = IN_CH + 3
WN_OUT = 16
NK = N * K
TW = 32
CM = LAST_CH * WN_OUT

NW = 32
CH = 512
NCHUNK = NK // CH
CPW_LO = NCHUNK // NW
REM = NCHUNK - CPW_LO * NW
NB = 4
GMAX = (CPW_LO + 1 + NB - 1) // NB


@functools.cache
def _make_sc_gather():
    mesh = plsc.VectorSubcoreMesh(core_axis_name="c", subcore_axis_name="s")

    scratch = (
        [pltpu.VMEM((CH,), jnp.int32) for _ in range(NB)]
        + [pltpu.VMEM((CH, TW), jnp.float32) for _ in range(NB)]
        + [pltpu.SemaphoreType.DMA for _ in range(3 * NB)]
    )

    @functools.partial(
        pl.kernel,
        mesh=mesh,
        out_type=jax.ShapeDtypeStruct((NK, TW), jnp.float32),
        scratch_types=scratch,
        compiler_params=pltpu.CompilerParams(use_tc_tiling_on_sc=False),
    )
    def _sc_gather(idx_hbm, tbl_hbm, out_hbm, *scr):
        idx_v = scr[0:NB]
        bt = scr[NB:2 * NB]
        isem = scr[2 * NB:3 * NB]
        gsem = scr[3 * NB:4 * NB]
        wsem = scr[4 * NB:5 * NB]

        wid = lax.axis_index("s") * 2 + lax.axis_index("c")
        nch = CPW_LO + (wid < REM).astype(jnp.int32)

        def off(j):
            return wid * CH + j * (NW * CH)

        for b in range(NB):
            pltpu.async_copy(idx_hbm.at[pl.ds(off(b), CH)], idx_v[b], isem[b])

        def body(g, carry):
            for b in range(NB):
                j = g * NB + b

                @pl.when(j < nch)
                def _():
                    @pl.when(j >= NB)
                    def _():
                        pltpu.make_async_copy(
                            bt[b], out_hbm.at[pl.ds(off(j - NB), CH)],
                            wsem[b]).wait()

                    pltpu.make_async_copy(
                        idx_hbm.at[pl.ds(off(j), CH)], idx_v[b],
                        isem[b]).wait()
                    pltpu.async_copy(tbl_hbm.at[idx_v[b]], bt[b], gsem[b])

            for b in range(NB):
                j = g * NB + b

                @pl.when(j < nch)
                def _():
                    pltpu.make_async_copy(
                        tbl_hbm.at[idx_v[b]], bt[b], gsem[b]).wait()
                    pltpu.async_copy(
                        bt[b], out_hbm.at[pl.ds(off(j), CH)], wsem[b])

                    @pl.when(j + NB < nch)
                    def _():
                        pltpu.async_copy(
                            idx_hbm.at[pl.ds(off(j + NB), CH)], idx_v[b],
                            isem[b])

            return carry

        lax.fori_loop(0, GMAX, body, 0)

        for b in range(NB):
            pltpu.make_async_copy(
                bt[b], out_hbm.at[pl.ds(off(0), CH)], wsem[b]).wait()

    return _sc_gather


def _np_blockdiag(blk, n):
    return np.kron(np.eye(n, dtype=np.float32), blk).astype(np.float32)


def _build_static():
    e32 = np.zeros((TW, CM), np.float32)
    for u in range(LAST_CH):
        for m in range(WN_OUT):
            e32[u, u * WN_OUT + m] = 1.0
    eq4 = np.zeros((4, CM), np.float32)
    for d in range(3):
        for m in range(WN_OUT):
            eq4[d, (IN_CH + d) * WN_OUT + m] = 1.0
    t16 = np.zeros((WN_OUT, CM), np.float32)
    for m in range(WN_OUT):
        for c in range(LAST_CH):
            t16[m, c * WN_OUT + m] = 1.0
    s32 = np.zeros((TW, 4), np.float32)
    for i in range(4):
        s32[IN_CH + i, i] = 1.0
    return (
        jnp.asarray(_np_blockdiag(e32, 4)),
        jnp.asarray(_np_blockdiag(eq4, 4)),
        jnp.asarray(_np_blockdiag(t16, 4)),
        jnp.asarray(_np_blockdiag(s32, 4)),
    )


def _tc_body(g_ref, q_ref, wt1_ref, wq1_ref, wt2_ref, wt3_ref,
             et_ref, eq_ref, tt_ref, s_ref, wlt_ref,
             b1t_ref, b2t_ref, b3t_ref, blt_ref, of_ref, ow_ref):
    P4 = q_ref.shape[0]
    Rp = K * P4
    gp = g_ref[...]
    gp2 = gp.reshape(Rp, 128)
    q = jnp.dot(q_ref[...], s_ref[...],
                preferred_element_type=jnp.float32)

    t1 = jnp.dot(q, wq1_ref[...], preferred_element_type=jnp.float32)
    a1 = jnp.dot(gp2, wt1_ref[...], preferred_element_type=jnp.float32)
    h = jnp.maximum(a1.reshape(K, P4, 32) - t1[None] + b1t_ref[...], 0.0)
    h = h.reshape(Rp, 32)
    h = jnp.maximum(
        jnp.dot(h, wt2_ref[...], preferred_element_type=jnp.float32)
        + b2t_ref[...], 0.0)
    wp = jnp.maximum(
        jnp.dot(h, wt3_ref[...], preferred_element_type=jnp.float32)
        + b3t_ref[...], 0.0)
    wp3 = wp.reshape(K, P4, 64)

    lp = jnp.dot(gp2, s_ref[...], preferred_element_type=jnp.float32)
    ow_ref[...] = lp.reshape(K, P4, 16) - q[None]

    qe = jnp.dot(q, eq_ref[...], preferred_element_type=jnp.float32)
    acc = None
    gpb = gp.astype(jnp.bfloat16)
    wpb = wp3.astype(jnp.bfloat16)
    etb = et_ref[...].astype(jnp.bfloat16)
    ttb = tt_ref[...].astype(jnp.bfloat16)
    for kg in range(4):
        gg = gpb[4 * kg:4 * kg + 4].reshape(4 * P4, 128)
        wg = wpb[4 * kg:4 * kg + 4].reshape(4 * P4, 64)
        grep = jnp.dot(gg, etb,
                       preferred_element_type=jnp.float32).reshape(
                           4, P4, 4 * CM) - qe[None]
        wrep = jnp.dot(wg, ttb,
                       preferred_element_type=jnp.float32).reshape(
                           4, P4, 4 * CM)
        z = grep * wrep
        sg = (z[0] + z[1]) + (z[2] + z[3])
        acc = sg if acc is None else acc + sg

    of_ref[...] = jnp.maximum(
        jnp.dot(acc, wlt_ref[...], preferred_element_type=jnp.float32)
        + blt_ref[...], 0.0)


def kernel(dense_xyz, dense_feats, nei_inds, W1, b1, W2, b2, W3, b3, Wl, bl):
    tbl = jnp.concatenate(
        [dense_feats[0], dense_xyz[0],
         jnp.zeros((N, TW - IN_CH - 3), jnp.float32)], axis=1)
    idx = nei_inds[0].T.reshape(NK)
    g = _make_sc_gather()(idx, tbl)
    g4 = g.reshape(K, N // 4, 128)

    tbl4 = tbl.reshape(N // 4, 128)

    bd = jax.scipy.linalg.block_diag
    W1blk = jnp.concatenate(
        [jnp.zeros((IN_CH, 8), jnp.float32), W1,
         jnp.zeros((TW - IN_CH - 3, 8), jnp.float32)], axis=0)
    W1q = jnp.concatenate([W1, jnp.zeros((1, 8), jnp.float32)], axis=0)
    Wt1 = bd(W1blk, W1blk, W1blk, W1blk)
    Wq1 = bd(W1q, W1q, W1q, W1q)
    Wt2 = bd(W2, W2, W2, W2)
    Wt3 = bd(W3, W3, W3, W3)
    Wlt = bd(Wl, Wl, Wl, Wl)
    Et, Eq, Tt, S = _build_static()
    b1t = jnp.tile(b1, 4).reshape(1, 32)
    b2t = jnp.tile(b2, 4).reshape(1, 32)
    b3t = jnp.tile(b3, 4).reshape(1, 64)
    blt = jnp.tile(bl, 4).reshape(1, 256)

    P = 800
    P4 = P // 4
    grid = (N // P,)
    full = lambda i: (0, 0)
    of4, ow = pl.pallas_call(
        _tc_body,
        grid=grid,
        in_specs=[
            pl.BlockSpec((K, P4, 128), lambda i: (0, i, 0)),
            pl.BlockSpec((P4, 128), lambda i: (i, 0)),
            pl.BlockSpec((128, 32), full),
            pl.BlockSpec((16, 32), full),
            pl.BlockSpec((32, 32), full),
            pl.BlockSpec((32, 64), full),
            pl.BlockSpec((128, 4 * CM), full),
            pl.BlockSpec((16, 4 * CM), full),
            pl.BlockSpec((64, 4 * CM), full),
            pl.BlockSpec((128, 16), full),
            pl.BlockSpec((4 * CM, 256), full),
            pl.BlockSpec((1, 32), full),
            pl.BlockSpec((1, 32), full),
            pl.BlockSpec((1, 64), full),
            pl.BlockSpec((1, 256), full),
        ],
        out_specs=[
            pl.BlockSpec((P4, 256), lambda i: (i, 0)),
            pl.BlockSpec((K, P4, 16), lambda i: (0, i, 0)),
        ],
        out_shape=[
            jax.ShapeDtypeStruct((N // 4, 256), jnp.float32),
            jax.ShapeDtypeStruct((K, N // 4, 16), jnp.float32),
        ],
    )(g4, tbl4, Wt1, Wq1, Wt2, Wt3, Et, Eq, Tt, S, Wlt,
      b1t, b2t, b3t, blt)
    of = of4.reshape(B, N, OUT_CH)
    wni = ow.reshape(K, N, 4)[:, :, :3].transpose(1, 0, 2).reshape(B, N, K, 3)
    return (of, wni)
